# baseline (speedup 1.0000x reference)
import functools

import jax
import jax.numpy as jnp
from jax import lax
from jax.experimental import pallas as pl
from jax.experimental.pallas import tpu as pltpu
from jax.experimental.pallas import tpu_sc as plsc

T = 16384
D = 4096
E = 64
N_GROUPS = 8
G = E // N_GROUPS
TOPK_GROUPS = 4
TOPK = 8
NSLOT = TOPK_GROUPS * G

BT = 1024
NW = 32
TPT = T // NW
L = 16
NEG = jnp.float32(-jnp.inf)


def _tc_body(x_ref, wt_ref, b_ref, sb_out_ref):
    s = jnp.dot(x_ref[...], wt_ref[...], preferred_element_type=jnp.float32)
    st = s.T
    m = jnp.max(st, axis=0, keepdims=True)
    e = jnp.exp(st - m)
    sb = e / jnp.sum(e, axis=0, keepdims=True) + b_ref[...]
    sb_out_ref[0, :, :] = sb[:, :TPT]
    sb_out_ref[1, :, :] = sb[:, TPT:]


def _sc_body(sb_hbm, bias1_hbm, w_hbm, i_hbm,
             sb_v, bias1_v, work_va, emap_va, work_vb, emap_vb, wout_v, iout_v):
    cid = lax.axis_index("c")
    sid = lax.axis_index("s")
    wid = sid * 2 + cid
    pltpu.sync_copy(sb_hbm.at[wid], sb_v)
    pltpu.sync_copy(bias1_hbm, bias1_v)
    lane = lax.iota(jnp.int32, L)

    def process16(t0, work_v, emap_v):
        sl = pl.ds(t0, L)
        tok = t0 + lane

        gsum = []
        for g in range(N_GROUPS):
            e0 = g * G
            a = sb_v[e0, sl]
            b = jnp.full((L,), NEG, jnp.float32)
            for kk in range(1, G):
                xk = sb_v[e0 + kk, sl]
                b = jnp.maximum(b, jnp.minimum(a, xk))
                a = jnp.maximum(a, xk)
            gsum.append(a + b)

        gs = gsum
        gsel = []
        for _ in range(TOPK_GROUPS):
            best = jnp.full((L,), NEG, jnp.float32)
            bidx = jnp.zeros((L,), jnp.int32)
            for g in range(N_GROUPS):
                cond = gs[g] > best
                best = jnp.where(cond, gs[g], best)
                bidx = jnp.where(cond, jnp.int32(g), bidx)
            gs = [jnp.where(bidx == g, NEG, gs[g]) for g in range(N_GROUPS)]
            gsel.append(bidx)

        for (i, k) in ((0, 1), (2, 3), (0, 2), (1, 3), (1, 2)):
            lo = jnp.minimum(gsel[i], gsel[k])
            hi = jnp.maximum(gsel[i], gsel[k])
            gsel[i], gsel[k] = lo, hi

        for r4 in range(TOPK_GROUPS):
            rowbase = gsel[r4] * G
            for kk in range(G):
                rows = rowbase + kk
                slot = r4 * G + kk
                v = plsc.load_gather(sb_v, [rows, tok])
                bits = plsc.bitcast(v, jnp.int32)
                low = jnp.where(v >= 0.0, jnp.int32(31 - slot), jnp.int32(slot))
                pv = plsc.bitcast((bits & jnp.int32(-32)) | low, jnp.float32)
                work_v[pl.ds(slot * L, L)] = pv
                emap_v[pl.ds(slot * L, L)] = rows

        for r in range(TOPK):
            vals = [work_v[pl.ds(s * L, L)] for s in range(NSLOT)]
            while len(vals) > 1:
                vals = [jnp.maximum(vals[2 * i], vals[2 * i + 1])
                        for i in range(len(vals) // 2)]
            best = vals[0]
            low5 = plsc.bitcast(best, jnp.int32) & jnp.int32(31)
            bslot = jnp.where(best >= 0.0, jnp.int32(31) - low5, low5)
            fidx = bslot * L + lane
            plsc.store_scatter(work_v, [fidx], jnp.full((L,), NEG, jnp.float32))
            eidx = plsc.load_gather(emap_v, [fidx])
            wval = (plsc.load_gather(sb_v, [eidx, tok])
                    - plsc.load_gather(bias1_v, [eidx]))
            plsc.store_scatter(iout_v, [tok * TOPK + r], eidx)
            plsc.store_scatter(wout_v, [tok * TOPK + r], wval)

    def group_body(j, carry):
        t0 = j * (2 * L)
        process16(t0, work_va, emap_va)
        process16(t0 + L, work_vb, emap_vb)
        return carry

    lax.fori_loop(0, TPT // (2 * L), group_body, jnp.int32(0))

    base = wid * TPT * TOPK
    pltpu.sync_copy(wout_v, w_hbm.at[pl.ds(base, TPT * TOPK)])
    pltpu.sync_copy(iout_v, i_hbm.at[pl.ds(base, TPT * TOPK)])


_sc_route = functools.partial(
    pl.kernel,
    out_type=[
        jax.ShapeDtypeStruct((T * TOPK,), jnp.float32),
        jax.ShapeDtypeStruct((T * TOPK,), jnp.int32),
    ],
    mesh=plsc.VectorSubcoreMesh(core_axis_name="c", subcore_axis_name="s"),
    compiler_params=pltpu.CompilerParams(needs_layout_passes=False),
    scratch_types=[
        pltpu.VMEM((E, TPT), jnp.float32),
        pltpu.VMEM((E,), jnp.float32),
        pltpu.VMEM((NSLOT * L,), jnp.float32),
        pltpu.VMEM((NSLOT * L,), jnp.int32),
        pltpu.VMEM((NSLOT * L,), jnp.float32),
        pltpu.VMEM((NSLOT * L,), jnp.int32),
        pltpu.VMEM((TPT * TOPK,), jnp.float32),
        pltpu.VMEM((TPT * TOPK,), jnp.int32),
    ],
)(_sc_body)


def kernel(x, weight, bias):
    wt = weight.T
    sb_chunks = pl.pallas_call(
        _tc_body,
        grid=(T // BT,),
        in_specs=[
            pl.BlockSpec((BT, D), lambda i: (i, 0)),
            pl.BlockSpec((D, E), lambda i: (0, 0)),
            pl.BlockSpec((E, 1), lambda i: (0, 0)),
        ],
        out_specs=pl.BlockSpec((BT // TPT, E, TPT), lambda i: (i, 0, 0)),
        out_shape=jax.ShapeDtypeStruct((NW, E, TPT), jnp.float32),
        compiler_params=pltpu.CompilerParams(
            dimension_semantics=("parallel",),
        ),
    )(x, wt, bias.reshape(E, 1))
    w_flat, i_flat = _sc_route(sb_chunks, bias)
    return w_flat.reshape(T, TOPK), i_flat.reshape(T, TOPK)


# speedup vs baseline: 1.2502x; 1.2502x over previous
import functools

import jax
import jax.numpy as jnp
from jax import lax
from jax.experimental import pallas as pl
from jax.experimental.pallas import tpu as pltpu
from jax.experimental.pallas import tpu_sc as plsc

T = 16384
D = 4096
E = 64
N_GROUPS = 8
G = E // N_GROUPS
TOPK_GROUPS = 4
TOPK = 8
NSLOT = TOPK_GROUPS * G

BT = 1024
NW = 32
NPHASE = 2
TH = T // NPHASE
TPT = TH // NW
L = 16
NEG = jnp.float32(-jnp.inf)


def _tc_body(x_ref, wt_ref, b_ref, sb_out_ref):
    s = jnp.dot(x_ref[...], wt_ref[...], preferred_element_type=jnp.float32)
    st = s.T
    m = jnp.max(st, axis=0, keepdims=True)
    e = jnp.exp(st - m)
    sb = e / jnp.sum(e, axis=0, keepdims=True) + b_ref[...]
    for c in range(BT // TPT):
        sb_out_ref[c, :, :] = sb[:, c * TPT:(c + 1) * TPT]


def _sc_body(sb_hbm, bias1_hbm, w_hbm, i_hbm,
             sb_v, bias1_v, work_v, emap_v, wout_v, iout_v):
    cid = lax.axis_index("c")
    sid = lax.axis_index("s")
    wid = sid * 2 + cid
    pltpu.sync_copy(sb_hbm.at[wid], sb_v)
    pltpu.sync_copy(bias1_hbm, bias1_v)
    lane = lax.iota(jnp.int32, L)

    def group_body(j, carry):
        t0 = j * L
        sl = pl.ds(t0, L)
        tok = t0 + lane

        gsum = []
        for g in range(N_GROUPS):
            e0 = g * G
            a = sb_v[e0, sl]
            b = jnp.full((L,), NEG, jnp.float32)
            for kk in range(1, G):
                xk = sb_v[e0 + kk, sl]
                b = jnp.maximum(b, jnp.minimum(a, xk))
                a = jnp.maximum(a, xk)
            gsum.append(a + b)

        gs = gsum
        gsel = []
        for _ in range(TOPK_GROUPS):
            best = jnp.full((L,), NEG, jnp.float32)
            bidx = jnp.zeros((L,), jnp.int32)
            for g in range(N_GROUPS):
                cond = gs[g] > best
                best = jnp.where(cond, gs[g], best)
                bidx = jnp.where(cond, jnp.int32(g), bidx)
            gs = [jnp.where(bidx == g, NEG, gs[g]) for g in range(N_GROUPS)]
            gsel.append(bidx)

        for (i, k) in ((0, 1), (2, 3), (0, 2), (1, 3), (1, 2)):
            lo = jnp.minimum(gsel[i], gsel[k])
            hi = jnp.maximum(gsel[i], gsel[k])
            gsel[i], gsel[k] = lo, hi

        for r4 in range(TOPK_GROUPS):
            rowbase = gsel[r4] * G
            for kk in range(G):
                rows = rowbase + kk
                slot = r4 * G + kk
                v = plsc.load_gather(sb_v, [rows, tok])
                bits = plsc.bitcast(v, jnp.int32)
                low = jnp.where(v >= 0.0, jnp.int32(31 - slot), jnp.int32(slot))
                pv = plsc.bitcast((bits & jnp.int32(-32)) | low, jnp.float32)
                work_v[pl.ds(slot * L, L)] = pv
                emap_v[pl.ds(slot * L, L)] = rows

        for r in range(TOPK):
            vals = [work_v[pl.ds(s * L, L)] for s in range(NSLOT)]
            while len(vals) > 1:
                vals = [jnp.maximum(vals[2 * i], vals[2 * i + 1])
                        for i in range(len(vals) // 2)]
            best = vals[0]
            low5 = plsc.bitcast(best, jnp.int32) & jnp.int32(31)
            bslot = jnp.where(best >= 0.0, jnp.int32(31) - low5, low5)
            fidx = bslot * L + lane
            plsc.store_scatter(work_v, [fidx], jnp.full((L,), NEG, jnp.float32))
            eidx = plsc.load_gather(emap_v, [fidx])
            wval = (plsc.load_gather(sb_v, [eidx, tok])
                    - plsc.load_gather(bias1_v, [eidx]))
            plsc.store_scatter(iout_v, [tok * TOPK + r], eidx)
            plsc.store_scatter(wout_v, [tok * TOPK + r], wval)
        return carry

    lax.fori_loop(0, TPT // L, group_body, jnp.int32(0))

    base = wid * TPT * TOPK
    pltpu.sync_copy(wout_v, w_hbm.at[pl.ds(base, TPT * TOPK)])
    pltpu.sync_copy(iout_v, i_hbm.at[pl.ds(base, TPT * TOPK)])


_sc_route = functools.partial(
    pl.kernel,
    out_type=[
        jax.ShapeDtypeStruct((TH * TOPK,), jnp.float32),
        jax.ShapeDtypeStruct((TH * TOPK,), jnp.int32),
    ],
    mesh=plsc.VectorSubcoreMesh(core_axis_name="c", subcore_axis_name="s"),
    compiler_params=pltpu.CompilerParams(needs_layout_passes=False),
    scratch_types=[
        pltpu.VMEM((E, TPT), jnp.float32),
        pltpu.VMEM((E,), jnp.float32),
        pltpu.VMEM((NSLOT * L,), jnp.float32),
        pltpu.VMEM((NSLOT * L,), jnp.int32),
        pltpu.VMEM((TPT * TOPK,), jnp.float32),
        pltpu.VMEM((TPT * TOPK,), jnp.int32),
    ],
)(_sc_body)


def _tc_half(x, wt, b2, off_blk):
    return pl.pallas_call(
        _tc_body,
        grid=(TH // BT,),
        in_specs=[
            pl.BlockSpec((BT, D), lambda i: (i + off_blk, 0)),
            pl.BlockSpec((D, E), lambda i: (0, 0)),
            pl.BlockSpec((E, 1), lambda i: (0, 0)),
        ],
        out_specs=pl.BlockSpec((BT // TPT, E, TPT), lambda i: (i, 0, 0)),
        out_shape=jax.ShapeDtypeStruct((NW, E, TPT), jnp.float32),
        compiler_params=pltpu.CompilerParams(
            dimension_semantics=("parallel",),
        ),
    )(x, wt, b2)


def kernel(x, weight, bias):
    wt = weight.T
    b2 = bias.reshape(E, 1)
    ws, is_ = [], []
    sb_halves = []
    for h in range(NPHASE):
        sb_halves.append(_tc_half(x, wt, b2, h * (TH // BT)))
    for h in range(NPHASE):
        w_flat, i_flat = _sc_route(sb_halves[h], bias)
        ws.append(w_flat.reshape(TH, TOPK))
        is_.append(i_flat.reshape(TH, TOPK))
    return jnp.concatenate(ws, axis=0), jnp.concatenate(is_, axis=0)
